# baseline (speedup 1.0000x reference)
import math

import jax
import jax.numpy as jnp
from jax.experimental import pallas as pl
from jax.experimental.pallas import tpu as pltpu

N = 10000
D = 128
DEG_MAX = 64
OMEGA = 0.01
K_FRAC = 0.5
B_IDX = 2500
BM = 200
NM = N // BM
SQRT_M = math.sqrt(128.0)


def _main_body(adj_ref, x_ref, deg_ref, w_ref, b_ref, pe_ref, wg_ref, bg_ref,
               wb_ref, bb_ref, wa_ref, wr_ref,
               out_ref, nrm_ref, film_ref,
               h_s, gt_s, bt_s, ft_s, kthr_s):
    m = pl.program_id(0)

    @pl.when(m == 0)
    def _prologue():
        h = jnp.dot(x_ref[...], w_ref[...], preferred_element_type=jnp.float32)
        h_s[...] = (h + b_ref[...]) * SQRT_M
        g = jnp.dot(pe_ref[...], wg_ref[...], preferred_element_type=jnp.float32) + bg_ref[...]
        g = jnp.where(g >= 0.0, g, 0.01 * g)
        bt = jnp.dot(pe_ref[...], wb_ref[...], preferred_element_type=jnp.float32) + bb_ref[...]
        bt = jnp.where(bt >= 0.0, bt, 0.01 * bt)
        gt_s[...] = g
        bt_s[...] = bt
        ft_s[...] = (jnp.sqrt(jnp.sum(g * g, axis=1, keepdims=True))
                     + jnp.sqrt(jnp.sum(bt * bt, axis=1, keepdims=True)))
        kthr_s[...] = (jnp.sum(deg_ref[...].astype(jnp.float32), keepdims=True)
                       .reshape(1, 1) * (K_FRAC / N))

    agg = jnp.dot(adj_ref[...], h_s[...], preferred_element_type=jnp.float32)
    degi = deg_ref[pl.ds(m * BM, BM), :]
    deg = degi.astype(jnp.float32)
    hm = h_s[pl.ds(m * BM, BM), :]
    inv = jnp.where(deg > 0.0, 1.0 / deg, 0.0)
    iv = agg * inv
    io = jax.lax.broadcasted_iota(jnp.int32, (BM, DEG_MAX), 1)
    oh = (degi == io).astype(jnp.float32)
    gamma = jnp.dot(oh, gt_s[...], preferred_element_type=jnp.float32)
    beta = jnp.dot(oh, bt_s[...], preferred_element_type=jnp.float32)
    g1 = gamma + 1.0
    ba = g1 * jnp.dot(iv, wa_ref[...], preferred_element_type=jnp.float32) + beta
    br = g1 * jnp.dot(iv, wr_ref[...], preferred_element_type=jnp.float32) + beta
    r = (deg < kthr_s[0, 0]).astype(jnp.float32)
    bias = OMEGA * (r * ba - (1.0 - r) * br)
    out_ref[...] = (agg + hm + bias) / (deg + 1.0)
    na = jnp.sqrt(jnp.sum(ba * ba, axis=1, keepdims=True))
    nr = jnp.sqrt(jnp.sum(br * br, axis=1, keepdims=True))
    nrm_ref[...] = r * na + (1.0 - r) * nr
    film_ref[...] = jnp.dot(oh, ft_s[...], preferred_element_type=jnp.float32)


def _loss_body(idx_ref, nrmt_ref, filmt_ref, lb_ref, lf_ref):
    idx = idx_ref[...]
    hi = idx // 100
    lo = idx - hi * 100
    io = jax.lax.broadcasted_iota(jnp.int32, (B_IDX, 100), 1)
    oh_hi = (hi == io).astype(jnp.float32)
    oh_lo = (lo == io).astype(jnp.float32)
    tb = jnp.dot(oh_hi, nrmt_ref[...], preferred_element_type=jnp.float32)
    tf = jnp.dot(oh_hi, filmt_ref[...], preferred_element_type=jnp.float32)
    lb_ref[...] = jnp.sum(tb * oh_lo, keepdims=True).reshape(1, 1) * (1.0 / B_IDX)
    lf_ref[...] = jnp.sum(tf * oh_lo, keepdims=True).reshape(1, 1) * (1.0 / B_IDX)


def kernel(x, adj, degree, idx, edge, W, b, W_gamma, W_beta, b_gamma, b_beta,
           W_add, W_rev, PE):
    f32 = jnp.float32
    pe64 = PE[:DEG_MAX]
    b2 = b.reshape(1, D)
    degi = degree.astype(jnp.int32)

    out, nrm, film = pl.pallas_call(
        _main_body,
        grid=(NM,),
        in_specs=[
            pl.BlockSpec((BM, N), lambda m: (m, 0)),
            pl.BlockSpec((N, D), lambda m: (0, 0)),
            pl.BlockSpec((N, 1), lambda m: (0, 0)),
            pl.BlockSpec((D, D), lambda m: (0, 0)),
            pl.BlockSpec((1, D), lambda m: (0, 0)),
            pl.BlockSpec((DEG_MAX, D), lambda m: (0, 0)),
            pl.BlockSpec((D, D), lambda m: (0, 0)),
            pl.BlockSpec((1, D), lambda m: (0, 0)),
            pl.BlockSpec((D, D), lambda m: (0, 0)),
            pl.BlockSpec((1, D), lambda m: (0, 0)),
            pl.BlockSpec((D, D), lambda m: (0, 0)),
            pl.BlockSpec((D, D), lambda m: (0, 0)),
        ],
        out_specs=[
            pl.BlockSpec((BM, D), lambda m: (m, 0)),
            pl.BlockSpec((BM, 1), lambda m: (m, 0)),
            pl.BlockSpec((BM, 1), lambda m: (m, 0)),
        ],
        out_shape=[
            jax.ShapeDtypeStruct((N, D), f32),
            jax.ShapeDtypeStruct((N, 1), f32),
            jax.ShapeDtypeStruct((N, 1), f32),
        ],
        scratch_shapes=[
            pltpu.VMEM((N, D), f32),
            pltpu.VMEM((DEG_MAX, D), f32),
            pltpu.VMEM((DEG_MAX, D), f32),
            pltpu.VMEM((DEG_MAX, 1), f32),
            pltpu.VMEM((1, 1), f32),
        ],
        compiler_params=pltpu.CompilerParams(
            dimension_semantics=("arbitrary",),
        ),
    )(adj, x, degi, W, b2, pe64, W_gamma, b_gamma, W_beta, b_beta, W_add, W_rev)

    idx2 = idx.reshape(B_IDX, 1).astype(jnp.int32)
    lb, lf = pl.pallas_call(
        _loss_body,
        out_shape=[
            jax.ShapeDtypeStruct((1, 1), f32),
            jax.ShapeDtypeStruct((1, 1), f32),
        ],
    )(idx2, nrm.reshape(100, 100), film.reshape(100, 100))

    return out, lb[0, 0], lf[0, 0]


# speedup vs baseline: 1.0364x; 1.0364x over previous
import math

import jax
import jax.numpy as jnp
from jax.experimental import pallas as pl
from jax.experimental.pallas import tpu as pltpu

N = 10000
D = 128
DEG_MAX = 64
OMEGA = 0.01
K_FRAC = 0.5
B_IDX = 2500
BM = 400
NM = N // BM
SQRT_M = math.sqrt(128.0)


def _main_body(adj_ref, x_ref, deg_ref, idx_ref, w_ref, b_ref, pe_ref,
               wg_ref, bg_ref, wb_ref, bb_ref, wa_ref, wr_ref,
               out_ref, lbf_ref,
               h_s, gt_s, bt_s, ft_s, kthr_s):
    m = pl.program_id(0)

    @pl.when(m == 0)
    def _prologue():
        h = jnp.dot(x_ref[...], w_ref[...], preferred_element_type=jnp.float32)
        h_s[...] = (h + b_ref[...]) * SQRT_M
        g = jnp.dot(pe_ref[...], wg_ref[...], preferred_element_type=jnp.float32) + bg_ref[...]
        g = jnp.where(g >= 0.0, g, 0.01 * g)
        bt = jnp.dot(pe_ref[...], wb_ref[...], preferred_element_type=jnp.float32) + bb_ref[...]
        bt = jnp.where(bt >= 0.0, bt, 0.01 * bt)
        gt_s[...] = g
        bt_s[...] = bt
        ft_s[...] = (jnp.sqrt(jnp.sum(g * g, axis=1, keepdims=True))
                     + jnp.sqrt(jnp.sum(bt * bt, axis=1, keepdims=True)))
        kthr_s[...] = (jnp.sum(deg_ref[...].astype(jnp.float32), keepdims=True)
                       .reshape(1, 1) * (K_FRAC / N))
        lbf_ref[...] = jnp.zeros_like(lbf_ref)

    agg = jnp.dot(adj_ref[...], h_s[...], preferred_element_type=jnp.float32)
    degi = deg_ref[pl.ds(m * BM, BM), :]
    deg = degi.astype(jnp.float32)
    hm = h_s[pl.ds(m * BM, BM), :]
    inv = jnp.where(deg > 0.0, 1.0 / deg, 0.0)
    iv = agg * inv
    io = jax.lax.broadcasted_iota(jnp.int32, (BM, DEG_MAX), 1)
    oh = (degi == io).astype(jnp.float32)
    gamma = jnp.dot(oh, gt_s[...], preferred_element_type=jnp.float32)
    beta = jnp.dot(oh, bt_s[...], preferred_element_type=jnp.float32)
    g1 = gamma + 1.0
    ba = g1 * jnp.dot(iv, wa_ref[...], preferred_element_type=jnp.float32) + beta
    br = g1 * jnp.dot(iv, wr_ref[...], preferred_element_type=jnp.float32) + beta
    r = (deg < kthr_s[0, 0]).astype(jnp.float32)
    bias = OMEGA * (r * ba - (1.0 - r) * br)
    out_ref[...] = (agg + hm + bias) / (deg + 1.0)
    na = jnp.sqrt(jnp.sum(ba * ba, axis=1, keepdims=True))
    nr = jnp.sqrt(jnp.sum(br * br, axis=1, keepdims=True))
    nrm = r * na + (1.0 - r) * nr
    film = jnp.dot(oh, ft_s[...], preferred_element_type=jnp.float32)

    sel = idx_ref[...] - m * BM
    iol = jax.lax.broadcasted_iota(jnp.int32, (B_IDX, BM), 1)
    ohl = (sel == iol).astype(jnp.float32)
    both = jnp.concatenate([nrm, film], axis=1)
    vals = jnp.dot(ohl, both, preferred_element_type=jnp.float32)
    lbf_ref[...] += jnp.sum(vals, axis=0, keepdims=True)

    @pl.when(m == NM - 1)
    def _finish():
        lbf_ref[...] = lbf_ref[...] * (1.0 / B_IDX)


def kernel(x, adj, degree, idx, edge, W, b, W_gamma, W_beta, b_gamma, b_beta,
           W_add, W_rev, PE):
    f32 = jnp.float32
    pe64 = PE[:DEG_MAX]
    b2 = b.reshape(1, D)
    degi = degree.astype(jnp.int32)
    idx2 = idx.reshape(B_IDX, 1).astype(jnp.int32)

    out, lbf = pl.pallas_call(
        _main_body,
        grid=(NM,),
        in_specs=[
            pl.BlockSpec((BM, N), lambda m: (m, 0)),
            pl.BlockSpec((N, D), lambda m: (0, 0)),
            pl.BlockSpec((N, 1), lambda m: (0, 0)),
            pl.BlockSpec((B_IDX, 1), lambda m: (0, 0)),
            pl.BlockSpec((D, D), lambda m: (0, 0)),
            pl.BlockSpec((1, D), lambda m: (0, 0)),
            pl.BlockSpec((DEG_MAX, D), lambda m: (0, 0)),
            pl.BlockSpec((D, D), lambda m: (0, 0)),
            pl.BlockSpec((1, D), lambda m: (0, 0)),
            pl.BlockSpec((D, D), lambda m: (0, 0)),
            pl.BlockSpec((1, D), lambda m: (0, 0)),
            pl.BlockSpec((D, D), lambda m: (0, 0)),
            pl.BlockSpec((D, D), lambda m: (0, 0)),
        ],
        out_specs=[
            pl.BlockSpec((BM, D), lambda m: (m, 0)),
            pl.BlockSpec((1, 2), lambda m: (0, 0)),
        ],
        out_shape=[
            jax.ShapeDtypeStruct((N, D), f32),
            jax.ShapeDtypeStruct((1, 2), f32),
        ],
        scratch_shapes=[
            pltpu.VMEM((N, D), f32),
            pltpu.VMEM((DEG_MAX, D), f32),
            pltpu.VMEM((DEG_MAX, D), f32),
            pltpu.VMEM((DEG_MAX, 1), f32),
            pltpu.VMEM((1, 1), f32),
        ],
        compiler_params=pltpu.CompilerParams(
            dimension_semantics=("arbitrary",),
        ),
    )(adj, x, degi, idx2, W, b2, pe64, W_gamma, b_gamma, W_beta, b_beta,
      W_add, W_rev)

    return out, lbf[0, 0], lbf[0, 1]


# speedup vs baseline: 1.0896x; 1.0514x over previous
import math

import jax
import jax.numpy as jnp
from jax.experimental import pallas as pl
from jax.experimental.pallas import tpu as pltpu

N = 10000
D = 128
DEG_MAX = 64
OMEGA = 0.01
K_FRAC = 0.5
B_IDX = 2500
BM = 400
NM = N // BM
SQRT_M = math.sqrt(128.0)


def _main_body(adj_ref, x_ref, deg_ref, w_ref, b_ref, pe_ref, wg_ref, bg_ref,
               wb_ref, bb_ref, wa_ref, wr_ref,
               out_ref, nrm_ref, film_ref,
               h_s, gt_s, bt_s, ft_s, kthr_s):
    m = pl.program_id(0)

    @pl.when(m == 0)
    def _prologue():
        h = jnp.dot(x_ref[...], w_ref[...], preferred_element_type=jnp.float32)
        h_s[...] = (h + b_ref[...]) * SQRT_M
        g = jnp.dot(pe_ref[...], wg_ref[...], preferred_element_type=jnp.float32) + bg_ref[...]
        g = jnp.where(g >= 0.0, g, 0.01 * g)
        bt = jnp.dot(pe_ref[...], wb_ref[...], preferred_element_type=jnp.float32) + bb_ref[...]
        bt = jnp.where(bt >= 0.0, bt, 0.01 * bt)
        gt_s[...] = g
        bt_s[...] = bt
        ft_s[...] = (jnp.sqrt(jnp.sum(g * g, axis=1, keepdims=True))
                     + jnp.sqrt(jnp.sum(bt * bt, axis=1, keepdims=True)))
        kthr_s[...] = (jnp.sum(deg_ref[...].astype(jnp.float32), keepdims=True)
                       .reshape(1, 1) * (K_FRAC / N))

    agg = jnp.dot(adj_ref[...], h_s[...], preferred_element_type=jnp.float32)
    degi = deg_ref[pl.ds(m * BM, BM), :]
    deg = degi.astype(jnp.float32)
    hm = h_s[pl.ds(m * BM, BM), :]
    inv = jnp.where(deg > 0.0, 1.0 / deg, 0.0)
    iv = agg * inv
    io = jax.lax.broadcasted_iota(jnp.int32, (BM, DEG_MAX), 1)
    oh = (degi == io).astype(jnp.float32)
    gamma = jnp.dot(oh, gt_s[...], preferred_element_type=jnp.float32)
    beta = jnp.dot(oh, bt_s[...], preferred_element_type=jnp.float32)
    g1 = gamma + 1.0
    ba = g1 * jnp.dot(iv, wa_ref[...], preferred_element_type=jnp.float32) + beta
    br = g1 * jnp.dot(iv, wr_ref[...], preferred_element_type=jnp.float32) + beta
    r = (deg < kthr_s[0, 0]).astype(jnp.float32)
    bias = OMEGA * (r * ba - (1.0 - r) * br)
    out_ref[...] = (agg + hm + bias) / (deg + 1.0)
    na = jnp.sqrt(jnp.sum(ba * ba, axis=1, keepdims=True))
    nr = jnp.sqrt(jnp.sum(br * br, axis=1, keepdims=True))
    nrm_ref[...] = r * na + (1.0 - r) * nr
    film_ref[...] = jnp.dot(oh, ft_s[...], preferred_element_type=jnp.float32)


def _loss_body(idx_ref, nrm_ref, film_ref, lb_ref, lf_ref):
    idx = idx_ref[...]
    hi = idx // 100
    lo = idx - hi * 100
    io = jax.lax.broadcasted_iota(jnp.int32, (B_IDX, 100), 1)
    oh_hi = (hi == io).astype(jnp.float32)
    oh_lo = (lo == io).astype(jnp.float32)
    nrmt = nrm_ref[...].reshape(100, 100)
    filmt = film_ref[...].reshape(100, 100)
    tb = jnp.dot(oh_hi, nrmt, preferred_element_type=jnp.float32)
    tf = jnp.dot(oh_hi, filmt, preferred_element_type=jnp.float32)
    lb_ref[...] = jnp.sum(tb * oh_lo, keepdims=True).reshape(1, 1) * (1.0 / B_IDX)
    lf_ref[...] = jnp.sum(tf * oh_lo, keepdims=True).reshape(1, 1) * (1.0 / B_IDX)


def kernel(x, adj, degree, idx, edge, W, b, W_gamma, W_beta, b_gamma, b_beta,
           W_add, W_rev, PE):
    f32 = jnp.float32
    pe64 = PE[:DEG_MAX]
    b2 = b.reshape(1, D)
    degi = degree.astype(jnp.int32)

    out, nrm, film = pl.pallas_call(
        _main_body,
        grid=(NM,),
        in_specs=[
            pl.BlockSpec((BM, N), lambda m: (m, 0)),
            pl.BlockSpec((N, D), lambda m: (0, 0)),
            pl.BlockSpec((N, 1), lambda m: (0, 0)),
            pl.BlockSpec((D, D), lambda m: (0, 0)),
            pl.BlockSpec((1, D), lambda m: (0, 0)),
            pl.BlockSpec((DEG_MAX, D), lambda m: (0, 0)),
            pl.BlockSpec((D, D), lambda m: (0, 0)),
            pl.BlockSpec((1, D), lambda m: (0, 0)),
            pl.BlockSpec((D, D), lambda m: (0, 0)),
            pl.BlockSpec((1, D), lambda m: (0, 0)),
            pl.BlockSpec((D, D), lambda m: (0, 0)),
            pl.BlockSpec((D, D), lambda m: (0, 0)),
        ],
        out_specs=[
            pl.BlockSpec((BM, D), lambda m: (m, 0)),
            pl.BlockSpec((BM, 1), lambda m: (m, 0)),
            pl.BlockSpec((BM, 1), lambda m: (m, 0)),
        ],
        out_shape=[
            jax.ShapeDtypeStruct((N, D), f32),
            jax.ShapeDtypeStruct((N, 1), f32),
            jax.ShapeDtypeStruct((N, 1), f32),
        ],
        scratch_shapes=[
            pltpu.VMEM((N, D), f32),
            pltpu.VMEM((DEG_MAX, D), f32),
            pltpu.VMEM((DEG_MAX, D), f32),
            pltpu.VMEM((DEG_MAX, 1), f32),
            pltpu.VMEM((1, 1), f32),
        ],
        compiler_params=pltpu.CompilerParams(
            dimension_semantics=("arbitrary",),
        ),
    )(adj, x, degi, W, b2, pe64, W_gamma, b_gamma, W_beta, b_beta, W_add, W_rev)

    idx2 = idx.reshape(B_IDX, 1).astype(jnp.int32)
    lb, lf = pl.pallas_call(
        _loss_body,
        out_shape=[
            jax.ShapeDtypeStruct((1, 1), f32),
            jax.ShapeDtypeStruct((1, 1), f32),
        ],
    )(idx2, nrm, film)

    return out, lb[0, 0], lf[0, 0]


# speedup vs baseline: 1.0917x; 1.0019x over previous
import math

import jax
import jax.numpy as jnp
from jax.experimental import pallas as pl
from jax.experimental.pallas import tpu as pltpu

N = 10000
D = 128
DEG_MAX = 64
OMEGA = 0.01
K_FRAC = 0.5
B_IDX = 2500
BM = 400
NM = N // BM
SQRT_M = math.sqrt(128.0)


def _main_body(adj_a_ref, adj_b_ref, x_ref, deg_ref, w_ref, b_ref, pe_ref,
               wg_ref, bg_ref, wb_ref, bb_ref, wa_ref, wr_ref,
               out_ref, nrm_ref, film_ref,
               h_s, gt_s, bt_s, ft_s, kthr_s):
    m = pl.program_id(0)

    @pl.when(m == 0)
    def _prologue():
        h = jnp.dot(x_ref[...], w_ref[...], preferred_element_type=jnp.float32)
        h_s[...] = (h + b_ref[...]) * SQRT_M
        g = jnp.dot(pe_ref[...], wg_ref[...], preferred_element_type=jnp.float32) + bg_ref[...]
        g = jnp.where(g >= 0.0, g, 0.01 * g)
        bt = jnp.dot(pe_ref[...], wb_ref[...], preferred_element_type=jnp.float32) + bb_ref[...]
        bt = jnp.where(bt >= 0.0, bt, 0.01 * bt)
        gt_s[...] = g
        bt_s[...] = bt
        ft_s[...] = (jnp.sqrt(jnp.sum(g * g, axis=1, keepdims=True))
                     + jnp.sqrt(jnp.sum(bt * bt, axis=1, keepdims=True)))
        kthr_s[...] = (jnp.sum(deg_ref[...].astype(jnp.float32), keepdims=True)
                       .reshape(1, 1) * (K_FRAC / N))

    agg = jnp.concatenate(
        [jnp.dot(adj_a_ref[...], h_s[...], preferred_element_type=jnp.float32),
         jnp.dot(adj_b_ref[...], h_s[...], preferred_element_type=jnp.float32)],
        axis=0)
    degi = deg_ref[pl.ds(m * BM, BM), :]
    deg = degi.astype(jnp.float32)
    hm = h_s[pl.ds(m * BM, BM), :]
    inv = jnp.where(deg > 0.0, 1.0 / deg, 0.0)
    iv = agg * inv
    io = jax.lax.broadcasted_iota(jnp.int32, (BM, DEG_MAX), 1)
    oh = (degi == io).astype(jnp.float32)
    gamma = jnp.dot(oh, gt_s[...], preferred_element_type=jnp.float32)
    beta = jnp.dot(oh, bt_s[...], preferred_element_type=jnp.float32)
    g1 = gamma + 1.0
    ba = g1 * jnp.dot(iv, wa_ref[...], preferred_element_type=jnp.float32) + beta
    br = g1 * jnp.dot(iv, wr_ref[...], preferred_element_type=jnp.float32) + beta
    r = (deg < kthr_s[0, 0]).astype(jnp.float32)
    bias = OMEGA * (r * ba - (1.0 - r) * br)
    out_ref[...] = (agg + hm + bias) / (deg + 1.0)
    na = jnp.sqrt(jnp.sum(ba * ba, axis=1, keepdims=True))
    nr = jnp.sqrt(jnp.sum(br * br, axis=1, keepdims=True))
    nrm_ref[...] = r * na + (1.0 - r) * nr
    film_ref[...] = jnp.dot(oh, ft_s[...], preferred_element_type=jnp.float32)


def _loss_body(idx_ref, nrm_ref, film_ref, lb_ref, lf_ref):
    idx = idx_ref[...]
    hi = idx // 100
    lo = idx - hi * 100
    io = jax.lax.broadcasted_iota(jnp.int32, (B_IDX, 100), 1)
    oh_hi = (hi == io).astype(jnp.float32)
    oh_lo = (lo == io).astype(jnp.float32)
    nrmt = nrm_ref[...].reshape(100, 100)
    filmt = film_ref[...].reshape(100, 100)
    tb = jnp.dot(oh_hi, nrmt, preferred_element_type=jnp.float32)
    tf = jnp.dot(oh_hi, filmt, preferred_element_type=jnp.float32)
    lb_ref[...] = jnp.sum(tb * oh_lo, keepdims=True).reshape(1, 1) * (1.0 / B_IDX)
    lf_ref[...] = jnp.sum(tf * oh_lo, keepdims=True).reshape(1, 1) * (1.0 / B_IDX)


def kernel(x, adj, degree, idx, edge, W, b, W_gamma, W_beta, b_gamma, b_beta,
           W_add, W_rev, PE):
    f32 = jnp.float32
    pe64 = PE[:DEG_MAX]
    b2 = b.reshape(1, D)
    degi = degree.astype(jnp.int32)

    out, nrm, film = pl.pallas_call(
        _main_body,
        grid=(NM,),
        in_specs=[
            pl.BlockSpec((BM // 2, N), lambda m: (2 * m, 0)),
            pl.BlockSpec((BM // 2, N), lambda m: (2 * m + 1, 0)),
            pl.BlockSpec((N, D), lambda m: (0, 0)),
            pl.BlockSpec((N, 1), lambda m: (0, 0)),
            pl.BlockSpec((D, D), lambda m: (0, 0)),
            pl.BlockSpec((1, D), lambda m: (0, 0)),
            pl.BlockSpec((DEG_MAX, D), lambda m: (0, 0)),
            pl.BlockSpec((D, D), lambda m: (0, 0)),
            pl.BlockSpec((1, D), lambda m: (0, 0)),
            pl.BlockSpec((D, D), lambda m: (0, 0)),
            pl.BlockSpec((1, D), lambda m: (0, 0)),
            pl.BlockSpec((D, D), lambda m: (0, 0)),
            pl.BlockSpec((D, D), lambda m: (0, 0)),
        ],
        out_specs=[
            pl.BlockSpec((BM, D), lambda m: (m, 0)),
            pl.BlockSpec((BM, 1), lambda m: (m, 0)),
            pl.BlockSpec((BM, 1), lambda m: (m, 0)),
        ],
        out_shape=[
            jax.ShapeDtypeStruct((N, D), f32),
            jax.ShapeDtypeStruct((N, 1), f32),
            jax.ShapeDtypeStruct((N, 1), f32),
        ],
        scratch_shapes=[
            pltpu.VMEM((N, D), f32),
            pltpu.VMEM((DEG_MAX, D), f32),
            pltpu.VMEM((DEG_MAX, D), f32),
            pltpu.VMEM((DEG_MAX, 1), f32),
            pltpu.VMEM((1, 1), f32),
        ],
        compiler_params=pltpu.CompilerParams(
            dimension_semantics=("arbitrary",),
        ),
    )(adj, adj, x, degi, W, b2, pe64, W_gamma, b_gamma, W_beta, b_beta,
      W_add, W_rev)

    idx2 = idx.reshape(B_IDX, 1).astype(jnp.int32)
    lb, lf = pl.pallas_call(
        _loss_body,
        out_shape=[
            jax.ShapeDtypeStruct((1, 1), f32),
            jax.ShapeDtypeStruct((1, 1), f32),
        ],
    )(idx2, nrm, film)

    return out, lb[0, 0], lf[0, 0]


# speedup vs baseline: 1.1297x; 1.0348x over previous
import math

import jax
import jax.numpy as jnp
from jax.experimental import pallas as pl
from jax.experimental.pallas import tpu as pltpu

N = 10000
D = 128
DEG_MAX = 64
OMEGA = 0.01
K_FRAC = 0.5
B_IDX = 2500
BM = 400
NM = N // BM
SQRT_M = math.sqrt(128.0)


def _main_body(adj_ref, x_ref, deg_ref, idx_ref, w_ref, b_ref, pe_ref,
               wg_ref, bg_ref, wb_ref, bb_ref, wa_ref, wr_ref,
               out_ref, lbf_ref,
               h_s, gt_s, bt_s, ft_s, kthr_s, nrm_s, film_s):
    m = pl.program_id(0)

    @pl.when(m == 0)
    def _prologue():
        h = jnp.dot(x_ref[...], w_ref[...], preferred_element_type=jnp.float32)
        h_s[...] = (h + b_ref[...]) * SQRT_M
        g = jnp.dot(pe_ref[...], wg_ref[...], preferred_element_type=jnp.float32) + bg_ref[...]
        g = jnp.where(g >= 0.0, g, 0.01 * g)
        bt = jnp.dot(pe_ref[...], wb_ref[...], preferred_element_type=jnp.float32) + bb_ref[...]
        bt = jnp.where(bt >= 0.0, bt, 0.01 * bt)
        gt_s[...] = g
        bt_s[...] = bt
        ft_s[...] = (jnp.sqrt(jnp.sum(g * g, axis=1, keepdims=True))
                     + jnp.sqrt(jnp.sum(bt * bt, axis=1, keepdims=True)))
        kthr_s[...] = (jnp.sum(deg_ref[...].astype(jnp.float32), keepdims=True)
                       .reshape(1, 1) * (K_FRAC / N))

    agg = jnp.dot(adj_ref[...], h_s[...], preferred_element_type=jnp.float32)
    degi = deg_ref[pl.ds(m * BM, BM), :]
    deg = degi.astype(jnp.float32)
    hm = h_s[pl.ds(m * BM, BM), :]
    inv = jnp.where(deg > 0.0, 1.0 / deg, 0.0)
    iv = agg * inv
    io = jax.lax.broadcasted_iota(jnp.int32, (BM, DEG_MAX), 1)
    oh = (degi == io).astype(jnp.float32)
    gamma = jnp.dot(oh, gt_s[...], preferred_element_type=jnp.float32)
    beta = jnp.dot(oh, bt_s[...], preferred_element_type=jnp.float32)
    g1 = gamma + 1.0
    ba = g1 * jnp.dot(iv, wa_ref[...], preferred_element_type=jnp.float32) + beta
    br = g1 * jnp.dot(iv, wr_ref[...], preferred_element_type=jnp.float32) + beta
    r = (deg < kthr_s[0, 0]).astype(jnp.float32)
    bias = OMEGA * (r * ba - (1.0 - r) * br)
    out_ref[...] = (agg + hm + bias) / (deg + 1.0)
    na = jnp.sqrt(jnp.sum(ba * ba, axis=1, keepdims=True))
    nr = jnp.sqrt(jnp.sum(br * br, axis=1, keepdims=True))
    nrm_s[pl.ds(m, 1), :] = (r * na + (1.0 - r) * nr).reshape(1, BM)
    film_s[pl.ds(m, 1), :] = jnp.dot(
        oh, ft_s[...], preferred_element_type=jnp.float32).reshape(1, BM)

    @pl.when(m == NM - 1)
    def _loss():
        idx = idx_ref[...]
        hi = idx // BM
        lo = idx - hi * BM
        ioh = jax.lax.broadcasted_iota(jnp.int32, (B_IDX, NM), 1)
        iol = jax.lax.broadcasted_iota(jnp.int32, (B_IDX, BM), 1)
        oh_hi = (hi == ioh).astype(jnp.float32)
        oh_lo = (lo == iol).astype(jnp.float32)
        tb = jnp.dot(oh_hi, nrm_s[...], preferred_element_type=jnp.float32)
        tf = jnp.dot(oh_hi, film_s[...], preferred_element_type=jnp.float32)
        lb = jnp.sum(tb * oh_lo, keepdims=True).reshape(1, 1)
        lf = jnp.sum(tf * oh_lo, keepdims=True).reshape(1, 1)
        lbf_ref[...] = jnp.concatenate([lb, lf], axis=1) * (1.0 / B_IDX)


def kernel(x, adj, degree, idx, edge, W, b, W_gamma, W_beta, b_gamma, b_beta,
           W_add, W_rev, PE):
    f32 = jnp.float32
    pe64 = PE[:DEG_MAX]
    b2 = b.reshape(1, D)
    degi = degree.astype(jnp.int32)
    idx2 = idx.reshape(B_IDX, 1).astype(jnp.int32)

    out, lbf = pl.pallas_call(
        _main_body,
        grid=(NM,),
        in_specs=[
            pl.BlockSpec((BM, N), lambda m: (m, 0)),
            pl.BlockSpec((N, D), lambda m: (0, 0)),
            pl.BlockSpec((N, 1), lambda m: (0, 0)),
            pl.BlockSpec((B_IDX, 1), lambda m: (0, 0)),
            pl.BlockSpec((D, D), lambda m: (0, 0)),
            pl.BlockSpec((1, D), lambda m: (0, 0)),
            pl.BlockSpec((DEG_MAX, D), lambda m: (0, 0)),
            pl.BlockSpec((D, D), lambda m: (0, 0)),
            pl.BlockSpec((1, D), lambda m: (0, 0)),
            pl.BlockSpec((D, D), lambda m: (0, 0)),
            pl.BlockSpec((1, D), lambda m: (0, 0)),
            pl.BlockSpec((D, D), lambda m: (0, 0)),
            pl.BlockSpec((D, D), lambda m: (0, 0)),
        ],
        out_specs=[
            pl.BlockSpec((BM, D), lambda m: (m, 0)),
            pl.BlockSpec((1, 2), lambda m: (0, 0)),
        ],
        out_shape=[
            jax.ShapeDtypeStruct((N, D), f32),
            jax.ShapeDtypeStruct((1, 2), f32),
        ],
        scratch_shapes=[
            pltpu.VMEM((N, D), f32),
            pltpu.VMEM((DEG_MAX, D), f32),
            pltpu.VMEM((DEG_MAX, D), f32),
            pltpu.VMEM((DEG_MAX, 1), f32),
            pltpu.VMEM((1, 1), f32),
            pltpu.VMEM((NM, BM), f32),
            pltpu.VMEM((NM, BM), f32),
        ],
        compiler_params=pltpu.CompilerParams(
            dimension_semantics=("arbitrary",),
        ),
    )(adj, x, degi, idx2, W, b2, pe64, W_gamma, b_gamma, W_beta, b_beta,
      W_add, W_rev)

    return out, lbf[0, 0], lbf[0, 1]
